# baseline (device time: 38764 ns/iter reference)
import jax
import jax.numpy as jnp
from jax import lax
from jax.experimental import pallas as pl
from jax.experimental.pallas import tpu as pltpu

N_DEV = 8
XOR_STEPS = (1, 3, 4)
N_LAYERS = 3
N_CHUNK = 2


def kernel(x, Win0, Wout0, Win1, Wout1, Win2, Wout2):
    b, d = x.shape
    h_dim = Win0.shape[1]
    w = d // N_CHUNK
    kw = d // N_CHUNK

    def body(x_ref, win0_ref, wout0_ref, win1_ref, wout1_ref,
             win2_ref, wout2_ref, out_ref, acc_ref, hn_ref, send_ref,
             recv_ref, win_buf, wout_buf, load_sems, send_sems, recv_sems):
        my = lax.axis_index("i")
        f32 = jnp.float32
        DEFAULT = lax.Precision.DEFAULT

        wins_hbm = [win0_ref, win1_ref, win2_ref]
        wouts_hbm = [wout0_ref, wout1_ref, wout2_ref]
        loads = []
        for l in range(N_LAYERS):
            cw = pltpu.make_async_copy(
                wins_hbm[l], win_buf.at[l], load_sems.at[2 * l])
            co = pltpu.make_async_copy(
                wouts_hbm[l], wout_buf.at[l], load_sems.at[2 * l + 1])
            cw.start()
            co.start()
            loads.append((cw, co))

        barrier = pltpu.get_barrier_semaphore()
        for s in XOR_STEPS:
            pl.semaphore_signal(
                barrier, inc=1,
                device_id=(my ^ s,), device_id_type=pl.DeviceIdType.MESH,
            )
        pl.semaphore_wait(barrier, len(XOR_STEPS))

        def cols(c):
            return pl.ds(c * w, w)

        def step_of(c, r):
            return XOR_STEPS[(r + c) % len(XOR_STEPS)]

        def slot_of(c, l, r):
            return (l * len(XOR_STEPS) + r) * N_CHUNK + c

        def make_rdma(c, l, r):
            slot = slot_of(c, l, r)
            return pltpu.make_async_remote_copy(
                src_ref=send_ref.at[c],
                dst_ref=recv_ref.at[slot],
                send_sem=send_sems.at[slot],
                recv_sem=recv_sems.at[slot],
                device_id=(my ^ step_of(c, r),),
                device_id_type=pl.DeviceIdType.MESH,
            )

        inflight = {}

        def launch_layer(l, h):
            for c in range(N_CHUNK):
                p = jnp.dot(h, wout_buf[l, :, c * w:(c + 1) * w],
                            precision=DEFAULT, preferred_element_type=f32)
                acc_ref[:, cols(c)] = p
                send_ref[c, :, :] = p.astype(jnp.bfloat16)
                rdma = make_rdma(c, l, 0)
                rdma.start()
                inflight[c] = rdma

        loads[0][0].wait()
        h = jnp.dot(x_ref[...], win_buf[0],
                    precision=DEFAULT, preferred_element_type=f32)
        h = jnp.maximum(h, 0.0)
        loads[0][1].wait()
        launch_layer(0, h)

        for l in range(N_LAYERS):
            for r in range(len(XOR_STEPS)):
                for c in range(N_CHUNK):
                    inflight[c].wait()
                    a = (acc_ref[:, cols(c)]
                         + recv_ref[slot_of(c, l, r)].astype(f32))
                    if r + 1 < len(XOR_STEPS):
                        acc_ref[:, cols(c)] = a
                        send_ref[c, :, :] = a.astype(jnp.bfloat16)
                        rdma = make_rdma(c, l, r + 1)
                        rdma.start()
                        inflight[c] = rdma
                    elif l + 1 < N_LAYERS:
                        if c == 0:
                            loads[l + 1][0].wait()
                            hn_ref[...] = jnp.dot(
                                a, win_buf[l + 1, 0:kw, :],
                                precision=DEFAULT,
                                preferred_element_type=f32)
                        else:
                            hn = hn_ref[...] + jnp.dot(
                                a, win_buf[l + 1, kw:2 * kw, :],
                                precision=DEFAULT,
                                preferred_element_type=f32)
                            h = jnp.maximum(hn, 0.0)
                            loads[l + 1][1].wait()
                            launch_layer(l + 1, h)
                    else:
                        out_ref[:, cols(c)] = a

    n_slots = N_LAYERS * len(XOR_STEPS) * N_CHUNK
    return pl.pallas_call(
        body,
        out_shape=jax.ShapeDtypeStruct((b, d), jnp.float32),
        in_specs=[pl.BlockSpec(memory_space=pltpu.VMEM)]
        + [pl.BlockSpec(memory_space=pltpu.MemorySpace.HBM)] * 6,
        out_specs=pl.BlockSpec(memory_space=pltpu.VMEM),
        scratch_shapes=[
            pltpu.VMEM((b, d), jnp.float32),
            pltpu.VMEM((b, h_dim), jnp.float32),
            pltpu.VMEM((N_CHUNK, b, w), jnp.bfloat16),
            pltpu.VMEM((n_slots, b, w), jnp.bfloat16),
            pltpu.VMEM((N_LAYERS, d, h_dim), jnp.float32),
            pltpu.VMEM((N_LAYERS, h_dim, d), jnp.float32),
            pltpu.SemaphoreType.DMA((2 * N_LAYERS,)),
            pltpu.SemaphoreType.DMA((n_slots,)),
            pltpu.SemaphoreType.DMA((n_slots,)),
        ],
        compiler_params=pltpu.CompilerParams(
            collective_id=0,
            vmem_limit_bytes=100 * 1024 * 1024,
        ),
    )(x, Win0, Wout0, Win1, Wout1, Win2, Wout2)


# device time: 38301 ns/iter; 1.0121x vs baseline; 1.0121x over previous
import jax
import jax.numpy as jnp
from jax import lax
from jax.experimental import pallas as pl
from jax.experimental.pallas import tpu as pltpu

N_DEV = 8
XOR_STEPS = (1, 3, 4)
N_LAYERS = 3
N_CHUNK = 2


def kernel(x, Win0, Wout0, Win1, Wout1, Win2, Wout2):
    b, d = x.shape
    h_dim = Win0.shape[1]
    w = d // N_CHUNK
    kw = d // N_CHUNK

    def body(x_ref, win0_ref, wout0_ref, win1_ref, wout1_ref,
             win2_ref, wout2_ref, out_ref, acc_ref, hn_ref, send_ref,
             recv_ref, win_buf, wout_buf, load_sems, send_sems, recv_sems):
        my = lax.axis_index("i")
        f32 = jnp.float32
        DEFAULT = lax.Precision.DEFAULT

        wins_hbm = [win0_ref, win1_ref, win2_ref]
        wouts_hbm = [wout0_ref, wout1_ref, wout2_ref]
        l0_loads = []
        for i in range(2):
            rs = pl.ds(i * (d // 2), d // 2)
            l0_loads.append(pltpu.make_async_copy(
                win0_ref.at[rs, :], win_buf.at[0, rs, :],
                load_sems.at[i]))
        for i in range(2):
            cs = pl.ds(i * w, w)
            l0_loads.append(pltpu.make_async_copy(
                wout0_ref.at[:, cs], wout_buf.at[0, :, cs],
                load_sems.at[2 + i]))
        for cp in l0_loads:
            cp.start()
        loads = [None]
        for l in range(1, N_LAYERS):
            cw = pltpu.make_async_copy(
                wins_hbm[l], win_buf.at[l], load_sems.at[2 * l + 2])
            co = pltpu.make_async_copy(
                wouts_hbm[l], wout_buf.at[l], load_sems.at[2 * l + 3])
            cw.start()
            co.start()
            loads.append((cw, co))

        barrier = pltpu.get_barrier_semaphore()
        for s in XOR_STEPS:
            pl.semaphore_signal(
                barrier, inc=1,
                device_id=(my ^ s,), device_id_type=pl.DeviceIdType.MESH,
            )
        pl.semaphore_wait(barrier, len(XOR_STEPS))

        def cols(c):
            return pl.ds(c * w, w)

        def step_of(c, r):
            return XOR_STEPS[(r + c) % len(XOR_STEPS)]

        def slot_of(c, l, r):
            return (l * len(XOR_STEPS) + r) * N_CHUNK + c

        def make_rdma(c, l, r):
            slot = slot_of(c, l, r)
            return pltpu.make_async_remote_copy(
                src_ref=send_ref.at[c],
                dst_ref=recv_ref.at[slot],
                send_sem=send_sems.at[slot],
                recv_sem=recv_sems.at[slot],
                device_id=(my ^ step_of(c, r),),
                device_id_type=pl.DeviceIdType.MESH,
            )

        inflight = {}

        def launch_layer(l, h):
            for c in range(N_CHUNK):
                p = jnp.dot(h, wout_buf[l, :, c * w:(c + 1) * w],
                            precision=DEFAULT, preferred_element_type=f32)
                acc_ref[:, cols(c)] = p
                send_ref[c, :, :] = p.astype(jnp.bfloat16)
                rdma = make_rdma(c, l, 0)
                rdma.start()
                inflight[c] = rdma

        l0_loads[0].wait()
        hn = jnp.dot(x_ref[:, 0:d // 2], win_buf[0, 0:d // 2, :],
                     precision=DEFAULT, preferred_element_type=f32)
        l0_loads[1].wait()
        hn = hn + jnp.dot(x_ref[:, d // 2:d], win_buf[0, d // 2:d, :],
                          precision=DEFAULT, preferred_element_type=f32)
        h = jnp.maximum(hn, 0.0)
        for c in range(N_CHUNK):
            l0_loads[2 + c].wait()
            p = jnp.dot(h, wout_buf[0, :, c * w:(c + 1) * w],
                        precision=DEFAULT, preferred_element_type=f32)
            acc_ref[:, cols(c)] = p
            send_ref[c, :, :] = p.astype(jnp.bfloat16)
            rdma = make_rdma(c, 0, 0)
            rdma.start()
            inflight[c] = rdma

        for l in range(N_LAYERS):
            for r in range(len(XOR_STEPS)):
                for c in range(N_CHUNK):
                    inflight[c].wait()
                    a = (acc_ref[:, cols(c)]
                         + recv_ref[slot_of(c, l, r)].astype(f32))
                    if r + 1 < len(XOR_STEPS):
                        acc_ref[:, cols(c)] = a
                        send_ref[c, :, :] = a.astype(jnp.bfloat16)
                        rdma = make_rdma(c, l, r + 1)
                        rdma.start()
                        inflight[c] = rdma
                    elif l + 1 < N_LAYERS:
                        if c == 0:
                            loads[l + 1][0].wait()
                            hn_ref[...] = jnp.dot(
                                a, win_buf[l + 1, 0:kw, :],
                                precision=DEFAULT,
                                preferred_element_type=f32)
                        else:
                            hn = hn_ref[...] + jnp.dot(
                                a, win_buf[l + 1, kw:2 * kw, :],
                                precision=DEFAULT,
                                preferred_element_type=f32)
                            h = jnp.maximum(hn, 0.0)
                            loads[l + 1][1].wait()
                            launch_layer(l + 1, h)
                    else:
                        out_ref[:, cols(c)] = a

    n_slots = N_LAYERS * len(XOR_STEPS) * N_CHUNK
    return pl.pallas_call(
        body,
        out_shape=jax.ShapeDtypeStruct((b, d), jnp.float32),
        in_specs=[pl.BlockSpec(memory_space=pltpu.VMEM)]
        + [pl.BlockSpec(memory_space=pltpu.MemorySpace.HBM)] * 6,
        out_specs=pl.BlockSpec(memory_space=pltpu.VMEM),
        scratch_shapes=[
            pltpu.VMEM((b, d), jnp.float32),
            pltpu.VMEM((b, h_dim), jnp.float32),
            pltpu.VMEM((N_CHUNK, b, w), jnp.bfloat16),
            pltpu.VMEM((n_slots, b, w), jnp.bfloat16),
            pltpu.VMEM((N_LAYERS, d, h_dim), jnp.float32),
            pltpu.VMEM((N_LAYERS, h_dim, d), jnp.float32),
            pltpu.SemaphoreType.DMA((2 * N_LAYERS + 2,)),
            pltpu.SemaphoreType.DMA((n_slots,)),
            pltpu.SemaphoreType.DMA((n_slots,)),
        ],
        compiler_params=pltpu.CompilerParams(
            collective_id=0,
            vmem_limit_bytes=100 * 1024 * 1024,
        ),
    )(x, Win0, Wout0, Win1, Wout1, Win2, Wout2)
